# baseline (speedup 1.0000x reference)
import functools

import jax
import jax.numpy as jnp
from jax import lax
from jax.experimental import pallas as pl
from jax.experimental.pallas import tpu as pltpu
from jax.experimental.pallas import tpu_sc as plsc

_NUM_CODEBOOKS = 8
_VOCAB = 1024
_D_IN = 64
_D_OUT = 64
_B, _T = 16, 2048
_N = _B * _T * _NUM_CODEBOOKS
_LANES = 128
_ROWS = _N // _LANES
_TCH = _T // 8


def _make_tc_prep(nw, chunks_per_w):
    def _tc_body(tokens_ref, table_ref, w_ref, b_ref, shifted_ref, p_ref):
        offs = (lax.broadcasted_iota(jnp.int32, (nw, chunks_per_w, _LANES), 2)
                % _NUM_CODEBOOKS) * _VOCAB
        shifted_ref[...] = tokens_ref[...] + offs
        p_ref[...] = jnp.dot(table_ref[...], w_ref[...],
                             preferred_element_type=jnp.float32) + b_ref[...]

    def prep(tokens3d, emb_table, W, b2d):
        return pl.pallas_call(
            _tc_body,
            out_shape=[
                jax.ShapeDtypeStruct((nw, chunks_per_w, _LANES), jnp.int32),
                jax.ShapeDtypeStruct((_NUM_CODEBOOKS * _VOCAB, _D_OUT), jnp.float32),
            ],
        )(tokens3d, emb_table, W, b2d)

    return prep


def _make_sc_gather(nw, chunks_per_w, num_cores):
    mesh = plsc.VectorSubcoreMesh(core_axis_name="c", subcore_axis_name="s")

    @functools.partial(
        pl.kernel,
        mesh=mesh,
        compiler_params=pltpu.CompilerParams(use_tc_tiling_on_sc=False),
        out_type=jax.ShapeDtypeStruct((_N, 2 * _D_OUT), jnp.float32),
        scratch_types=[
            pltpu.VMEM((chunks_per_w, _LANES), jnp.int32),
            pltpu.VMEM((_LANES, _D_OUT), jnp.float32),
            pltpu.VMEM((_LANES, _D_OUT), jnp.float32),
            pltpu.SemaphoreType.DMA,
            pltpu.SemaphoreType.DMA,
        ],
    )
    def sc_gather(p_hbm, idx_hbm, out_hbm, idx_v, rows_a, rows_b, sem_a, sem_b):
        wid = lax.axis_index("s") * num_cores + lax.axis_index("c")
        base = wid * (chunks_per_w * _LANES)
        pltpu.sync_copy(idx_hbm.at[wid], idx_v)

        pltpu.async_copy(p_hbm.at[idx_v.at[0]], rows_a, sem_a)

        def body(j, _):
            c0 = 2 * j
            pltpu.async_copy(p_hbm.at[idx_v.at[c0 + 1]], rows_b, sem_b)
            pltpu.make_async_copy(p_hbm.at[idx_v.at[c0]], rows_a, sem_a).wait()
            pltpu.sync_copy(rows_a,
                            out_hbm.at[pl.ds(base + c0 * _LANES, _LANES),
                                       pl.ds(0, _D_OUT)])

            @pl.when(c0 + 2 < chunks_per_w)
            def _():
                pltpu.async_copy(p_hbm.at[idx_v.at[c0 + 2]], rows_a, sem_a)

            pltpu.make_async_copy(p_hbm.at[idx_v.at[c0 + 1]], rows_b, sem_b).wait()
            pltpu.sync_copy(rows_b,
                            out_hbm.at[pl.ds(base + (c0 + 1) * _LANES, _LANES),
                                       pl.ds(0, _D_OUT)])
            return 0

        lax.fori_loop(0, chunks_per_w // 2, body, 0)

    return sc_gather


def _transpose_body(a_ref, x_ref):
    x = a_ref[...][:, :_D_OUT].reshape(_TCH, _NUM_CODEBOOKS, _D_OUT)
    x_ref[0] = jnp.transpose(x, (1, 2, 0))


def _transpose_finisher(a2d):
    return pl.pallas_call(
        _transpose_body,
        grid=(_B, _T // _TCH),
        in_specs=[pl.BlockSpec((_TCH * _NUM_CODEBOOKS, 2 * _D_OUT),
                               lambda i, j: (i * (_T // _TCH) + j, 0))],
        out_specs=pl.BlockSpec((1, _NUM_CODEBOOKS, _D_OUT, _TCH),
                               lambda i, j: (i, 0, 0, j)),
        out_shape=jax.ShapeDtypeStruct((_B, _NUM_CODEBOOKS, _D_OUT, _T),
                                       jnp.float32),
    )(a2d)


def kernel(in_tokens, emb_table, W, b):
    info = plsc.get_sparse_core_info()
    nw = info.num_cores * info.num_subcores
    chunks_per_w = _ROWS // nw
    tokens3d = in_tokens.reshape(nw, chunks_per_w, _LANES)
    prep = _make_tc_prep(nw, chunks_per_w)
    idx3d, proj = prep(tokens3d, emb_table, W, b.reshape(1, _D_OUT))
    sc_gather = _make_sc_gather(nw, chunks_per_w, info.num_cores)
    a2d = sc_gather(proj, idx3d)
    x = _transpose_finisher(a2d)
    return jnp.transpose(x, (0, 3, 1, 2))


# speedup vs baseline: 2.8842x; 2.8842x over previous
import functools

import jax
import jax.numpy as jnp
from jax import lax
from jax.experimental import pallas as pl
from jax.experimental.pallas import tpu as pltpu
from jax.experimental.pallas import tpu_sc as plsc

_NUM_CODEBOOKS = 8
_VOCAB = 1024
_D_IN = 64
_D_OUT = 64
_B, _T = 16, 2048
_N = _B * _T * _NUM_CODEBOOKS
_LANES = 128
_ROWS = _N // _LANES
_RING = 4


def _make_tc_prep(nw, chunks_per_w):
    def _tc_body(tokens_ref, table_ref, w_ref, b_ref, shifted_ref, p_ref):
        offs = (lax.broadcasted_iota(jnp.int32, (nw, chunks_per_w, _LANES), 2)
                % _NUM_CODEBOOKS) * _VOCAB
        shifted_ref[...] = tokens_ref[...] + offs
        p_ref[...] = jnp.dot(table_ref[...], w_ref[...],
                             preferred_element_type=jnp.float32) + b_ref[...]

    def prep(tokens3d, emb_table, W, b2d):
        return pl.pallas_call(
            _tc_body,
            out_shape=[
                jax.ShapeDtypeStruct((nw, chunks_per_w, _LANES), jnp.int32),
                jax.ShapeDtypeStruct((_NUM_CODEBOOKS * _VOCAB, _D_OUT), jnp.float32),
            ],
        )(tokens3d, emb_table, W, b2d)

    return prep


def _make_sc_gather(nw, chunks_per_w, num_cores):
    mesh = plsc.VectorSubcoreMesh(core_axis_name="c", subcore_axis_name="s")

    @functools.partial(
        pl.kernel,
        mesh=mesh,
        compiler_params=pltpu.CompilerParams(use_tc_tiling_on_sc=False),
        out_type=jax.ShapeDtypeStruct((_N, _D_OUT), jnp.float32),
        scratch_types=[
            pltpu.VMEM((chunks_per_w, _LANES), jnp.int32),
            pltpu.VMEM((_RING, _LANES, _D_OUT), jnp.float32),
            pltpu.SemaphoreType.DMA,
            pltpu.SemaphoreType.DMA,
        ],
    )
    def sc_gather(p_hbm, idx_hbm, out_hbm, idx_v, bufs, gsem, wsem):
        wid = lax.axis_index("s") * num_cores + lax.axis_index("c")
        base = wid * (chunks_per_w * _LANES)
        pltpu.sync_copy(idx_hbm.at[wid], idx_v)

        for q in range(_RING):
            pltpu.async_copy(p_hbm.at[idx_v.at[q]], bufs.at[q], gsem)

        n_groups = chunks_per_w // _RING

        def body(j, _):
            c0 = _RING * j
            for q in range(_RING):
                pltpu.make_async_copy(
                    p_hbm.at[idx_v.at[c0 + q]], bufs.at[q], gsem).wait()
                pltpu.async_copy(
                    bufs.at[q],
                    out_hbm.at[pl.ds(base + (c0 + q) * _LANES, _LANES)],
                    wsem)

            @pl.when(j < n_groups - 1)
            def _():
                for q in range(_RING):
                    pltpu.make_async_copy(
                        bufs.at[q],
                        out_hbm.at[pl.ds(base + (c0 + q) * _LANES, _LANES)],
                        wsem).wait()
                    pltpu.async_copy(
                        p_hbm.at[idx_v.at[c0 + _RING + q]], bufs.at[q], gsem)
            return 0

        lax.fori_loop(0, n_groups, body, 0)

        last = chunks_per_w - _RING
        for q in range(_RING):
            pltpu.make_async_copy(
                bufs.at[q],
                out_hbm.at[pl.ds(base + (last + q) * _LANES, _LANES)],
                wsem).wait()

    return sc_gather


def _transpose_body(a_ref, x_ref):
    x = a_ref[0]
    for c in range(_NUM_CODEBOOKS):
        x_ref[0, c] = jnp.transpose(x[:, c * _D_OUT:(c + 1) * _D_OUT])


def _transpose_finisher(a3d):
    return pl.pallas_call(
        _transpose_body,
        grid=(_B,),
        in_specs=[pl.BlockSpec((1, _T, _NUM_CODEBOOKS * _D_OUT),
                               lambda i: (i, 0, 0))],
        out_specs=pl.BlockSpec((1, _NUM_CODEBOOKS, _D_OUT, _T),
                               lambda i: (i, 0, 0, 0)),
        out_shape=jax.ShapeDtypeStruct((_B, _NUM_CODEBOOKS, _D_OUT, _T),
                                       jnp.float32),
    )(a3d)


def kernel(in_tokens, emb_table, W, b):
    info = plsc.get_sparse_core_info()
    nw = info.num_cores * info.num_subcores
    chunks_per_w = _ROWS // nw
    tokens3d = in_tokens.reshape(nw, chunks_per_w, _LANES)
    prep = _make_tc_prep(nw, chunks_per_w)
    idx3d, proj = prep(tokens3d, emb_table, W, b.reshape(1, _D_OUT))
    sc_gather = _make_sc_gather(nw, chunks_per_w, info.num_cores)
    out = sc_gather(proj, idx3d)
    a3d = out.reshape(_B, _T, _NUM_CODEBOOKS * _D_OUT)
    x = _transpose_finisher(a3d)
    return jnp.transpose(x, (0, 3, 1, 2))
